# baseline (speedup 1.0000x reference)
import functools

import jax
import jax.numpy as jnp
from jax import lax
from jax.experimental import pallas as pl
from jax.experimental.pallas import tpu as pltpu
from jax.experimental.pallas import tpu_sc as plsc

N = 10000
E = 160000
DIM = 128
NB = 8
RC = 5.0
BE = 640
NP = 5
EP = E // NP
BPP = EP // BE

_NC, _NS = 2, 16
_NW = _NC * _NS
CH = 128
CP = EP // CH
TPW = -(-CP // _NW)


def _sc_gather(table, idxp):
    mesh = plsc.VectorSubcoreMesh(core_axis_name="c", subcore_axis_name="s")
    out = jax.ShapeDtypeStruct((EP, DIM), jnp.float32)

    @functools.partial(
        pl.kernel,
        mesh=mesh,
        out_type=[out, out],
        scratch_types=[
            pltpu.VMEM((2, CH), jnp.int32),
            [[pltpu.VMEM((CH, DIM), jnp.float32) for _ in range(2)]
             for _ in range(2)],
            pltpu.SemaphoreType.DMA,
            [pltpu.SemaphoreType.DMA for _ in range(2)],
        ],
    )
    def gather_kernel(tab_hbm, idx_hbm, g0, g1, idx_v, rows, gsem, wsems):
        outs = (g0, g1)
        wid = lax.axis_index("s") * _NC + lax.axis_index("c")

        def wcopies(t):
            b = t % 2
            chunk = t * _NW + wid
            return [pltpu.make_async_copy(
                rows[b][a], outs[a].at[pl.ds(chunk * CH, CH)], wsems[b])
                for a in range(2)]

        for t in range(TPW):
            chunk = t * _NW + wid

            @pl.when(chunk < CP)
            def _():
                if t >= 2:
                    for c in wcopies(t - 2):
                        c.wait()
                pltpu.sync_copy(idx_hbm.at[pl.ds(2 * chunk, 2)], idx_v)
                b = t % 2
                cps = [
                    pltpu.async_copy(tab_hbm.at[idx_v.at[a]],
                                     rows[b][a], gsem)
                    for a in range(2)
                ]
                for c in cps:
                    c.wait()
                for c in wcopies(t):
                    c.start()

        for u in range(max(0, TPW - 3), TPW):
            skipped = (u * _NW + wid < CP)
            if u + 2 < TPW:
                skipped = skipped & ((u + 2) * _NW + wid >= CP)

            @pl.when(skipped)
            def _():
                for c in wcopies(u):
                    c.wait()

    return gather_kernel(table, idxp)


def _unpack_pair(packed):
    iv = lax.bitcast_convert_type(packed, jnp.uint32)
    hi = lax.bitcast_convert_type(iv & jnp.uint32(0xFFFF0000), jnp.float32)
    lo = lax.bitcast_convert_type(iv << 16, jnp.float32)
    return hi, lo


def _tc_body(evt_ref, g01_ref, g23_ref, e0_ref, e1_ref,
             wr000, wr110, wr011, wr101, wm01, wm02, wm1c, wm1g, wg,
             out0_ref, out1_ref):
    f32 = jnp.float32
    dimn = (((0,), (0,)), ((), ()))
    dott = lambda a, b: lax.dot_general(a, b, dimn, preferred_element_type=f32)
    dot = lambda a, b: jnp.dot(a, b, preferred_element_type=f32)

    evt = evt_ref[...]
    d2 = jnp.sum(evt * evt, axis=0, keepdims=True)
    d = jnp.sqrt(d2)
    rinv = 1.0 / (d + 1e-9)
    rt = evt * rinv
    centers = lax.broadcasted_iota(jnp.int32, (NB, 1), 0).astype(f32) \
        * (RC / (NB - 1))
    fc = 0.5 * (jnp.cos(jnp.pi * jnp.clip(d * (1.0 / RC), 0.0, 1.0)) + 1.0)
    rbft = jnp.exp(-2.0 * (d - centers) ** 2) * fc

    f000 = dott(rbft, wr000[...])
    f101 = dott(rbft, wr101[...])
    f110r = [dott(rbft * rt[a:a + 1], wr110[...]) for a in range(3)]
    f011r = [dott(rbft * rt[a:a + 1], wr011[...]) for a in range(3)]

    x0, x1a = _unpack_pair(g01_ref[...])
    x1b, x1c = _unpack_pair(g23_ref[...])
    x1 = (x1a, x1b, x1c)
    m0 = f000 * x0 + f110r[0] * x1[0] + f110r[1] * x1[1] + f110r[2] * x1[2]
    m1 = [f101 * x1[a] + f011r[a] * x0 for a in range(3)]

    s0 = dot(m0, wm01[...]) + dot(m0 * m0, wm02[...])
    gmix = dot(m0, wm1g[...])
    m1v = jnp.concatenate(m1, axis=0)
    s1v = dot(m1v, wm1c[...])

    n0 = s0 * jax.nn.sigmoid(s0)
    gate = jax.nn.sigmoid(dot(s0, wg[...]))
    out0_ref[...] = e0_ref[...] + n0
    for a in range(3):
        n1a = (s1v[a * BE:(a + 1) * BE] + gmix * m1[a]) * gate
        out1_ref[a] = e1_ref[a] + n1a


def _tc_body_acc(acc0_ref, acc1_ref, *args):
    del acc0_ref, acc1_ref
    _tc_body(*args)


def _tc_piece(p, accs, evt, g01, g23, edge_0, e1v, weights):
    row_f = lambda i, p=p: (p * BPP + i, 0)
    row_p = lambda i: (i, 0)
    comp = lambda i, p=p: (0, p * BPP + i, 0)
    fixed = lambda i: (0, 0)
    hbm = pl.BlockSpec(memory_space=pltpu.MemorySpace.HBM)
    in_specs = [
        pl.BlockSpec((3, BE), lambda i, p=p: (0, p * BPP + i)),
        pl.BlockSpec((BE, DIM), row_p),
        pl.BlockSpec((BE, DIM), row_p),
        pl.BlockSpec((BE, DIM), row_f),
        pl.BlockSpec((3, BE, DIM), comp),
    ] + [pl.BlockSpec((NB, DIM), fixed)] * 4 + \
        [pl.BlockSpec((DIM, DIM), fixed)] * 5
    operands = (evt, g01, g23, edge_0, e1v) + tuple(weights)
    kwargs = {}
    body = _tc_body
    if accs is not None:
        in_specs = [hbm, hbm] + in_specs
        operands = accs + operands
        kwargs["input_output_aliases"] = {0: 0, 1: 1}
        body = _tc_body_acc
    return pl.pallas_call(
        body,
        grid=(BPP,),
        in_specs=in_specs,
        out_specs=[
            pl.BlockSpec((BE, DIM), row_f),
            pl.BlockSpec((3, BE, DIM), comp),
        ],
        out_shape=[
            jax.ShapeDtypeStruct((E, DIM), jnp.float32),
            jax.ShapeDtypeStruct((3, E, DIM), jnp.float32),
        ],
        **kwargs,
    )(*operands)


def kernel(node_0, node_1, edge_0, edge_1, edge_index, edge_vec,
           Wr_000, Wr_110, Wr_011, Wr_101, Wm0_1, Wm0_2, Wm1_c, Wm1_g, Wg):
    idx = edge_index[0].astype(jnp.int32)
    off = jnp.array([0, N], dtype=jnp.int32).reshape(1, 2, 1)
    slabs = jnp.transpose(node_1, (2, 0, 1))
    e1v = jnp.transpose(edge_1, (2, 0, 1))
    evt = jnp.transpose(edge_vec)
    weights = (Wr_000, Wr_110, Wr_011, Wr_101, Wm0_1, Wm0_2, Wm1_c, Wm1_g, Wg)

    def _pack(hi, lo):
        hb = lax.bitcast_convert_type(hi.astype(jnp.bfloat16), jnp.uint16)
        lb = lax.bitcast_convert_type(lo.astype(jnp.bfloat16), jnp.uint16)
        w = (hb.astype(jnp.uint32) << 16) | lb.astype(jnp.uint32)
        return lax.bitcast_convert_type(w, jnp.float32)

    table = jnp.concatenate(
        [_pack(node_0, slabs[0]), _pack(slabs[1], slabs[2])], axis=0)

    gs = []
    for p in range(NP):
        idxp = (idx[p * EP:(p + 1) * EP].reshape(CP, 1, CH) + off) \
            .reshape(2 * CP, CH)
        gs.append(_sc_gather(table, idxp))
    accs = None
    for p in range(NP):
        g01, g23 = gs[p]
        out = _tc_piece(p, accs, evt, g01, g23, edge_0, e1v, weights)
        accs = tuple(out)
    out0, out1v = accs
    return (out0, jnp.transpose(out1v, (1, 2, 0)))


# speedup vs baseline: 1.0009x; 1.0009x over previous
import functools

import jax
import jax.numpy as jnp
from jax import lax
from jax.experimental import pallas as pl
from jax.experimental.pallas import tpu as pltpu
from jax.experimental.pallas import tpu_sc as plsc

N = 10000
E = 160000
DIM = 128
NB = 8
RC = 5.0
BE = 640
NP = 5
EP = E // NP
BPP = EP // BE

_NC, _NS = 2, 16
_NW = _NC * _NS
CH = 128
CP = EP // CH
TPW = -(-CP // _NW)


def _sc_gather(table, idxp):
    mesh = plsc.VectorSubcoreMesh(core_axis_name="c", subcore_axis_name="s")
    out = jax.ShapeDtypeStruct((EP, DIM), jnp.float32)

    @functools.partial(
        pl.kernel,
        mesh=mesh,
        out_type=[out, out],
        scratch_types=[
            pltpu.VMEM((2, CH), jnp.int32),
            [[pltpu.VMEM((CH, DIM), jnp.float32) for _ in range(2)]
             for _ in range(2)],
            pltpu.SemaphoreType.DMA,
            [pltpu.SemaphoreType.DMA for _ in range(2)],
        ],
    )
    def gather_kernel(tab_hbm, idx_hbm, g0, g1, idx_v, rows, gsem, wsems):
        outs = (g0, g1)
        wid = lax.axis_index("s") * _NC + lax.axis_index("c")

        def wcopies(t):
            b = t % 2
            chunk = t * _NW + wid
            return [pltpu.make_async_copy(
                rows[b][a], outs[a].at[pl.ds(chunk * CH, CH)], wsems[b])
                for a in range(2)]

        for t in range(TPW):
            chunk = t * _NW + wid

            @pl.when(chunk < CP)
            def _():
                if t >= 2:
                    for c in wcopies(t - 2):
                        c.wait()
                pltpu.sync_copy(idx_hbm.at[pl.ds(2 * chunk, 2)], idx_v)
                b = t % 2
                cps = [
                    pltpu.async_copy(
                        tab_hbm.at[idx_v.at[a, pl.ds(h * (CH // 2), CH // 2)]],
                        rows[b][a].at[pl.ds(h * (CH // 2), CH // 2)], gsem)
                    for a in range(2) for h in range(2)
                ]
                for c in cps:
                    c.wait()
                for c in wcopies(t):
                    c.start()

        for u in range(max(0, TPW - 3), TPW):
            skipped = (u * _NW + wid < CP)
            if u + 2 < TPW:
                skipped = skipped & ((u + 2) * _NW + wid >= CP)

            @pl.when(skipped)
            def _():
                for c in wcopies(u):
                    c.wait()

    return gather_kernel(table, idxp)


def _unpack_pair(packed):
    iv = lax.bitcast_convert_type(packed, jnp.uint32)
    hi = lax.bitcast_convert_type(iv & jnp.uint32(0xFFFF0000), jnp.float32)
    lo = lax.bitcast_convert_type(iv << 16, jnp.float32)
    return hi, lo


def _tc_body(evt_ref, g01_ref, g23_ref, e0_ref, e1_ref,
             wr000, wr110, wr011, wr101, wm01, wm02, wm1c, wm1g, wg,
             out0_ref, out1_ref):
    f32 = jnp.float32
    dimn = (((0,), (0,)), ((), ()))
    dott = lambda a, b: lax.dot_general(a, b, dimn, preferred_element_type=f32)
    dot = lambda a, b: jnp.dot(a, b, preferred_element_type=f32)

    evt = evt_ref[...]
    d2 = jnp.sum(evt * evt, axis=0, keepdims=True)
    d = jnp.sqrt(d2)
    rinv = 1.0 / (d + 1e-9)
    rt = evt * rinv
    centers = lax.broadcasted_iota(jnp.int32, (NB, 1), 0).astype(f32) \
        * (RC / (NB - 1))
    fc = 0.5 * (jnp.cos(jnp.pi * jnp.clip(d * (1.0 / RC), 0.0, 1.0)) + 1.0)
    rbft = jnp.exp(-2.0 * (d - centers) ** 2) * fc

    f000 = dott(rbft, wr000[...])
    f101 = dott(rbft, wr101[...])
    f110r = [dott(rbft * rt[a:a + 1], wr110[...]) for a in range(3)]
    f011r = [dott(rbft * rt[a:a + 1], wr011[...]) for a in range(3)]

    x0, x1a = _unpack_pair(g01_ref[...])
    x1b, x1c = _unpack_pair(g23_ref[...])
    x1 = (x1a, x1b, x1c)
    m0 = f000 * x0 + f110r[0] * x1[0] + f110r[1] * x1[1] + f110r[2] * x1[2]
    m1 = [f101 * x1[a] + f011r[a] * x0 for a in range(3)]

    s0 = dot(m0, wm01[...]) + dot(m0 * m0, wm02[...])
    gmix = dot(m0, wm1g[...])
    m1v = jnp.concatenate(m1, axis=0)
    s1v = dot(m1v, wm1c[...])

    n0 = s0 * jax.nn.sigmoid(s0)
    gate = jax.nn.sigmoid(dot(s0, wg[...]))
    out0_ref[...] = e0_ref[...] + n0
    for a in range(3):
        n1a = (s1v[a * BE:(a + 1) * BE] + gmix * m1[a]) * gate
        out1_ref[a] = e1_ref[a] + n1a


def _tc_body_acc(acc0_ref, acc1_ref, *args):
    del acc0_ref, acc1_ref
    _tc_body(*args)


def _tc_piece(p, accs, evt, g01, g23, edge_0, e1v, weights):
    row_f = lambda i, p=p: (p * BPP + i, 0)
    row_p = lambda i: (i, 0)
    comp = lambda i, p=p: (0, p * BPP + i, 0)
    fixed = lambda i: (0, 0)
    hbm = pl.BlockSpec(memory_space=pltpu.MemorySpace.HBM)
    in_specs = [
        pl.BlockSpec((3, BE), lambda i, p=p: (0, p * BPP + i)),
        pl.BlockSpec((BE, DIM), row_p),
        pl.BlockSpec((BE, DIM), row_p),
        pl.BlockSpec((BE, DIM), row_f),
        pl.BlockSpec((3, BE, DIM), comp),
    ] + [pl.BlockSpec((NB, DIM), fixed)] * 4 + \
        [pl.BlockSpec((DIM, DIM), fixed)] * 5
    operands = (evt, g01, g23, edge_0, e1v) + tuple(weights)
    kwargs = {}
    body = _tc_body
    if accs is not None:
        in_specs = [hbm, hbm] + in_specs
        operands = accs + operands
        kwargs["input_output_aliases"] = {0: 0, 1: 1}
        body = _tc_body_acc
    return pl.pallas_call(
        body,
        grid=(BPP,),
        in_specs=in_specs,
        out_specs=[
            pl.BlockSpec((BE, DIM), row_f),
            pl.BlockSpec((3, BE, DIM), comp),
        ],
        out_shape=[
            jax.ShapeDtypeStruct((E, DIM), jnp.float32),
            jax.ShapeDtypeStruct((3, E, DIM), jnp.float32),
        ],
        **kwargs,
    )(*operands)


def kernel(node_0, node_1, edge_0, edge_1, edge_index, edge_vec,
           Wr_000, Wr_110, Wr_011, Wr_101, Wm0_1, Wm0_2, Wm1_c, Wm1_g, Wg):
    idx = edge_index[0].astype(jnp.int32)
    off = jnp.array([0, N], dtype=jnp.int32).reshape(1, 2, 1)
    slabs = jnp.transpose(node_1, (2, 0, 1))
    e1v = jnp.transpose(edge_1, (2, 0, 1))
    evt = jnp.transpose(edge_vec)
    weights = (Wr_000, Wr_110, Wr_011, Wr_101, Wm0_1, Wm0_2, Wm1_c, Wm1_g, Wg)

    def _pack(hi, lo):
        hb = lax.bitcast_convert_type(hi.astype(jnp.bfloat16), jnp.uint16)
        lb = lax.bitcast_convert_type(lo.astype(jnp.bfloat16), jnp.uint16)
        w = (hb.astype(jnp.uint32) << 16) | lb.astype(jnp.uint32)
        return lax.bitcast_convert_type(w, jnp.float32)

    table = jnp.concatenate(
        [_pack(node_0, slabs[0]), _pack(slabs[1], slabs[2])], axis=0)

    gs = []
    for p in range(NP):
        idxp = (idx[p * EP:(p + 1) * EP].reshape(CP, 1, CH) + off) \
            .reshape(2 * CP, CH)
        gs.append(_sc_gather(table, idxp))
    accs = None
    for p in range(NP):
        g01, g23 = gs[p]
        out = _tc_piece(p, accs, evt, g01, g23, edge_0, e1v, weights)
        accs = tuple(out)
    out0, out1v = accs
    return (out0, jnp.transpose(out1v, (1, 2, 0)))


# speedup vs baseline: 1.0029x; 1.0020x over previous
import functools

import jax
import jax.numpy as jnp
from jax import lax
from jax.experimental import pallas as pl
from jax.experimental.pallas import tpu as pltpu
from jax.experimental.pallas import tpu_sc as plsc

N = 10000
E = 160000
DIM = 128
NB = 8
RC = 5.0
BE = 640
NP = 5
EP = E // NP
BPP = EP // BE

_NC, _NS = 2, 16
_NW = _NC * _NS
CH = 128
CP = EP // CH
TPW = -(-CP // _NW)


def _sc_gather(table, idxp):
    mesh = plsc.VectorSubcoreMesh(core_axis_name="c", subcore_axis_name="s")
    out = jax.ShapeDtypeStruct((EP, DIM), jnp.float32)

    @functools.partial(
        pl.kernel,
        mesh=mesh,
        out_type=[out, out],
        scratch_types=[
            pltpu.VMEM((2, CH), jnp.int32),
            [[pltpu.VMEM((CH, DIM), jnp.float32) for _ in range(2)]
             for _ in range(2)],
            pltpu.SemaphoreType.DMA,
            [pltpu.SemaphoreType.DMA for _ in range(2)],
        ],
    )
    def gather_kernel(tab_hbm, idx_hbm, g0, g1, idx_v, rows, gsem, wsems):
        outs = (g0, g1)
        wid = lax.axis_index("s") * _NC + lax.axis_index("c")

        def wcopies(t):
            b = t % 2
            chunk = t * _NW + wid
            return [pltpu.make_async_copy(
                rows[b][a], outs[a].at[pl.ds(chunk * CH, CH)], wsems[b])
                for a in range(2)]

        for t in range(TPW):
            chunk = t * _NW + wid

            @pl.when(chunk < CP)
            def _():
                if t >= 2:
                    for c in wcopies(t - 2):
                        c.wait()
                pltpu.sync_copy(idx_hbm.at[pl.ds(2 * chunk, 2)], idx_v)
                b = t % 2
                cps = [
                    pltpu.async_copy(
                        tab_hbm.at[idx_v.at[a, pl.ds(h * (CH // 2), CH // 2)]],
                        rows[b][a].at[pl.ds(h * (CH // 2), CH // 2)], gsem)
                    for a in range(2) for h in range(2)
                ]
                for c in cps:
                    c.wait()
                for c in wcopies(t):
                    c.start()

        for u in range(max(0, TPW - 3), TPW):
            skipped = (u * _NW + wid < CP)
            if u + 2 < TPW:
                skipped = skipped & ((u + 2) * _NW + wid >= CP)

            @pl.when(skipped)
            def _():
                for c in wcopies(u):
                    c.wait()

    return gather_kernel(table, idxp)


def _unpack_pair(packed):
    iv = lax.bitcast_convert_type(packed, jnp.uint32)
    hi = lax.bitcast_convert_type(iv & jnp.uint32(0xFFFF0000), jnp.float32)
    lo = lax.bitcast_convert_type(iv << 16, jnp.float32)
    return hi, lo


def _tc_body(evt_ref, g01_ref, g23_ref, e0_ref, e1_ref,
             wr000, wr110, wr011, wr101, wm01, wm02, wm1c, wm1g, wg,
             out0_ref, out1_ref):
    f32 = jnp.float32
    bf16 = jnp.bfloat16
    dimn = (((0,), (0,)), ((), ()))
    dott = lambda a, b: lax.dot_general(a, b, dimn, preferred_element_type=f32)
    dot = lambda a, b: jnp.dot(a.astype(bf16), b, preferred_element_type=f32)

    evt = evt_ref[...]
    d2 = jnp.sum(evt * evt, axis=0, keepdims=True)
    d = jnp.sqrt(d2)
    rinv = 1.0 / (d + 1e-9)
    rt = evt * rinv
    centers = lax.broadcasted_iota(jnp.int32, (NB, 1), 0).astype(f32) \
        * (RC / (NB - 1))
    fc = 0.5 * (jnp.cos(jnp.pi * jnp.clip(d * (1.0 / RC), 0.0, 1.0)) + 1.0)
    rbft = jnp.exp(-2.0 * (d - centers) ** 2) * fc

    f000 = dott(rbft, wr000[...])
    f101 = dott(rbft, wr101[...])
    f110r = [dott(rbft * rt[a:a + 1], wr110[...]) for a in range(3)]
    f011r = [dott(rbft * rt[a:a + 1], wr011[...]) for a in range(3)]

    x0, x1a = _unpack_pair(g01_ref[...])
    x1b, x1c = _unpack_pair(g23_ref[...])
    x1 = (x1a, x1b, x1c)
    m0 = f000 * x0 + f110r[0] * x1[0] + f110r[1] * x1[1] + f110r[2] * x1[2]
    m1 = [f101 * x1[a] + f011r[a] * x0 for a in range(3)]

    s0 = dot(m0, wm01[...]) + dot(m0 * m0, wm02[...])
    gmix = dot(m0, wm1g[...])
    s1 = [dot(m1[a], wm1c[...]) for a in range(3)]

    n0 = s0 * jax.nn.sigmoid(s0)
    gate = jax.nn.sigmoid(dot(s0, wg[...]))
    out0_ref[...] = e0_ref[...] + n0
    for a in range(3):
        n1a = (s1[a] + gmix * m1[a]) * gate
        out1_ref[a] = e1_ref[a] + n1a


def _tc_body_acc(acc0_ref, acc1_ref, *args):
    del acc0_ref, acc1_ref
    _tc_body(*args)


def _tc_piece(p, accs, evt, g01, g23, edge_0, e1v, weights):
    row_f = lambda i, p=p: (p * BPP + i, 0)
    row_p = lambda i: (i, 0)
    comp = lambda i, p=p: (0, p * BPP + i, 0)
    fixed = lambda i: (0, 0)
    hbm = pl.BlockSpec(memory_space=pltpu.MemorySpace.HBM)
    in_specs = [
        pl.BlockSpec((3, BE), lambda i, p=p: (0, p * BPP + i)),
        pl.BlockSpec((BE, DIM), row_p),
        pl.BlockSpec((BE, DIM), row_p),
        pl.BlockSpec((BE, DIM), row_f),
        pl.BlockSpec((3, BE, DIM), comp),
    ] + [pl.BlockSpec((NB, DIM), fixed)] * 4 + \
        [pl.BlockSpec((DIM, DIM), fixed)] * 5
    operands = (evt, g01, g23, edge_0, e1v) + tuple(weights)
    kwargs = {}
    body = _tc_body
    if accs is not None:
        in_specs = [hbm, hbm] + in_specs
        operands = accs + operands
        kwargs["input_output_aliases"] = {0: 0, 1: 1}
        body = _tc_body_acc
    return pl.pallas_call(
        body,
        grid=(BPP,),
        in_specs=in_specs,
        out_specs=[
            pl.BlockSpec((BE, DIM), row_f),
            pl.BlockSpec((3, BE, DIM), comp),
        ],
        out_shape=[
            jax.ShapeDtypeStruct((E, DIM), jnp.float32),
            jax.ShapeDtypeStruct((3, E, DIM), jnp.float32),
        ],
        **kwargs,
    )(*operands)


def kernel(node_0, node_1, edge_0, edge_1, edge_index, edge_vec,
           Wr_000, Wr_110, Wr_011, Wr_101, Wm0_1, Wm0_2, Wm1_c, Wm1_g, Wg):
    idx = edge_index[0].astype(jnp.int32)
    off = jnp.array([0, N], dtype=jnp.int32).reshape(1, 2, 1)
    slabs = jnp.transpose(node_1, (2, 0, 1))
    e1v = jnp.transpose(edge_1, (2, 0, 1))
    evt = jnp.transpose(edge_vec)
    bf = jnp.bfloat16
    weights = (Wr_000, Wr_110, Wr_011, Wr_101,
               Wm0_1.astype(bf), Wm0_2.astype(bf), Wm1_c.astype(bf),
               Wm1_g.astype(bf), Wg.astype(bf))

    def _pack(hi, lo):
        hb = lax.bitcast_convert_type(hi.astype(jnp.bfloat16), jnp.uint16)
        lb = lax.bitcast_convert_type(lo.astype(jnp.bfloat16), jnp.uint16)
        w = (hb.astype(jnp.uint32) << 16) | lb.astype(jnp.uint32)
        return lax.bitcast_convert_type(w, jnp.float32)

    table = jnp.concatenate(
        [_pack(node_0, slabs[0]), _pack(slabs[1], slabs[2])], axis=0)

    gs = []
    for p in range(NP):
        idxp = (idx[p * EP:(p + 1) * EP].reshape(CP, 1, CH) + off) \
            .reshape(2 * CP, CH)
        gs.append(_sc_gather(table, idxp))
    accs = None
    for p in range(NP):
        g01, g23 = gs[p]
        out = _tc_piece(p, accs, evt, g01, g23, edge_0, e1v, weights)
        accs = tuple(out)
    out0, out1v = accs
    return (out0, jnp.transpose(out1v, (1, 2, 0)))


# speedup vs baseline: 1.1634x; 1.1600x over previous
import functools

import jax
import jax.numpy as jnp
from jax import lax
from jax.experimental import pallas as pl
from jax.experimental.pallas import tpu as pltpu
from jax.experimental.pallas import tpu_sc as plsc

N = 10000
E = 160000
DIM = 128
NB = 8
RC = 5.0
BE = 1280
NP = 5
EP = E // NP
BPP = EP // BE

_NC, _NS = 2, 16
_NW = _NC * _NS
CH = 128
CP = EP // CH
TPW = -(-CP // _NW)


def _sc_gather(table, idxp):
    mesh = plsc.VectorSubcoreMesh(core_axis_name="c", subcore_axis_name="s")
    out = jax.ShapeDtypeStruct((EP, DIM), jnp.float32)

    @functools.partial(
        pl.kernel,
        mesh=mesh,
        out_type=[out, out],
        scratch_types=[
            pltpu.VMEM((2, CH), jnp.int32),
            [[pltpu.VMEM((CH, DIM), jnp.float32) for _ in range(2)]
             for _ in range(2)],
            pltpu.SemaphoreType.DMA,
            [pltpu.SemaphoreType.DMA for _ in range(2)],
        ],
    )
    def gather_kernel(tab_hbm, idx_hbm, g0, g1, idx_v, rows, gsem, wsems):
        outs = (g0, g1)
        wid = lax.axis_index("s") * _NC + lax.axis_index("c")

        def wcopies(t):
            b = t % 2
            chunk = t * _NW + wid
            return [pltpu.make_async_copy(
                rows[b][a], outs[a].at[pl.ds(chunk * CH, CH)], wsems[b])
                for a in range(2)]

        for t in range(TPW):
            chunk = t * _NW + wid

            @pl.when(chunk < CP)
            def _():
                if t >= 2:
                    for c in wcopies(t - 2):
                        c.wait()
                pltpu.sync_copy(idx_hbm.at[pl.ds(2 * chunk, 2)], idx_v)
                b = t % 2
                cps = [
                    pltpu.async_copy(
                        tab_hbm.at[idx_v.at[a, pl.ds(h * (CH // 2), CH // 2)]],
                        rows[b][a].at[pl.ds(h * (CH // 2), CH // 2)], gsem)
                    for a in range(2) for h in range(2)
                ]
                for c in cps:
                    c.wait()
                for c in wcopies(t):
                    c.start()

        for u in range(max(0, TPW - 3), TPW):
            skipped = (u * _NW + wid < CP)
            if u + 2 < TPW:
                skipped = skipped & ((u + 2) * _NW + wid >= CP)

            @pl.when(skipped)
            def _():
                for c in wcopies(u):
                    c.wait()

    return gather_kernel(table, idxp)


def _unpack_pair(packed):
    iv = lax.bitcast_convert_type(packed, jnp.uint32)
    hi = lax.bitcast_convert_type(iv & jnp.uint32(0xFFFF0000), jnp.float32)
    lo = lax.bitcast_convert_type(iv << 16, jnp.float32)
    return hi, lo


def _tc_body(evt_ref, g01_ref, g23_ref, e0_ref, e1_ref,
             wr000, wr110, wr011, wr101, wm01, wm02, wm1c, wm1g, wg,
             out0_ref, out1_ref):
    f32 = jnp.float32
    bf16 = jnp.bfloat16
    dimn = (((0,), (0,)), ((), ()))
    dott = lambda a, b: lax.dot_general(a, b, dimn, preferred_element_type=f32)
    dot = lambda a, b: jnp.dot(a.astype(bf16), b, preferred_element_type=f32)

    evt = evt_ref[...]
    d2 = jnp.sum(evt * evt, axis=0, keepdims=True)
    d = jnp.sqrt(d2)
    rinv = 1.0 / (d + 1e-9)
    rt = evt * rinv
    centers = lax.broadcasted_iota(jnp.int32, (NB, 1), 0).astype(f32) \
        * (RC / (NB - 1))
    fc = 0.5 * (jnp.cos(jnp.pi * jnp.clip(d * (1.0 / RC), 0.0, 1.0)) + 1.0)
    rbft = jnp.exp(-2.0 * (d - centers) ** 2) * fc

    f000 = dott(rbft, wr000[...])
    f101 = dott(rbft, wr101[...])
    f110r = [dott(rbft * rt[a:a + 1], wr110[...]) for a in range(3)]
    f011r = [dott(rbft * rt[a:a + 1], wr011[...]) for a in range(3)]

    x0, x1a = _unpack_pair(g01_ref[...])
    x1b, x1c = _unpack_pair(g23_ref[...])
    x1 = (x1a, x1b, x1c)
    m0 = f000 * x0 + f110r[0] * x1[0] + f110r[1] * x1[1] + f110r[2] * x1[2]
    m1 = [f101 * x1[a] + f011r[a] * x0 for a in range(3)]

    s0 = dot(m0, wm01[...]) + dot(m0 * m0, wm02[...])
    gmix = dot(m0, wm1g[...])
    s1 = [dot(m1[a], wm1c[...]) for a in range(3)]

    n0 = s0 * jax.nn.sigmoid(s0)
    gate = jax.nn.sigmoid(dot(s0, wg[...]))
    out0_ref[...] = e0_ref[...] + n0
    for a in range(3):
        n1a = (s1[a] + gmix * m1[a]) * gate
        out1_ref[a] = e1_ref[a] + n1a


def _tc_body_acc(acc0_ref, acc1_ref, *args):
    del acc0_ref, acc1_ref
    _tc_body(*args)


def _tc_piece(p, accs, evt, g01, g23, edge_0, e1v, weights):
    row_f = lambda i, p=p: (p * BPP + i, 0)
    row_p = lambda i: (i, 0)
    comp = lambda i, p=p: (0, p * BPP + i, 0)
    fixed = lambda i: (0, 0)
    hbm = pl.BlockSpec(memory_space=pltpu.MemorySpace.HBM)
    in_specs = [
        pl.BlockSpec((3, BE), lambda i, p=p: (0, p * BPP + i)),
        pl.BlockSpec((BE, DIM), row_p),
        pl.BlockSpec((BE, DIM), row_p),
        pl.BlockSpec((BE, DIM), row_f),
        pl.BlockSpec((3, BE, DIM), comp),
    ] + [pl.BlockSpec((NB, DIM), fixed)] * 4 + \
        [pl.BlockSpec((DIM, DIM), fixed)] * 5
    operands = (evt, g01, g23, edge_0, e1v) + tuple(weights)
    kwargs = {}
    body = _tc_body
    if accs is not None:
        in_specs = [hbm, hbm] + in_specs
        operands = accs + operands
        kwargs["input_output_aliases"] = {0: 0, 1: 1}
        body = _tc_body_acc
    return pl.pallas_call(
        body,
        grid=(BPP,),
        in_specs=in_specs,
        out_specs=[
            pl.BlockSpec((BE, DIM), row_f),
            pl.BlockSpec((3, BE, DIM), comp),
        ],
        out_shape=[
            jax.ShapeDtypeStruct((E, DIM), jnp.float32),
            jax.ShapeDtypeStruct((3, E, DIM), jnp.float32),
        ],
        **kwargs,
    )(*operands)


def kernel(node_0, node_1, edge_0, edge_1, edge_index, edge_vec,
           Wr_000, Wr_110, Wr_011, Wr_101, Wm0_1, Wm0_2, Wm1_c, Wm1_g, Wg):
    idx = edge_index[0].astype(jnp.int32)
    off = jnp.array([0, N], dtype=jnp.int32).reshape(1, 2, 1)
    slabs = jnp.transpose(node_1, (2, 0, 1))
    e1v = jnp.transpose(edge_1, (2, 0, 1))
    evt = jnp.transpose(edge_vec)
    bf = jnp.bfloat16
    weights = (Wr_000, Wr_110, Wr_011, Wr_101,
               Wm0_1.astype(bf), Wm0_2.astype(bf), Wm1_c.astype(bf),
               Wm1_g.astype(bf), Wg.astype(bf))

    def _pack(hi, lo):
        hb = lax.bitcast_convert_type(hi.astype(jnp.bfloat16), jnp.uint16)
        lb = lax.bitcast_convert_type(lo.astype(jnp.bfloat16), jnp.uint16)
        w = (hb.astype(jnp.uint32) << 16) | lb.astype(jnp.uint32)
        return lax.bitcast_convert_type(w, jnp.float32)

    table = jnp.concatenate(
        [_pack(node_0, slabs[0]), _pack(slabs[1], slabs[2])], axis=0)

    gs = []
    for p in range(NP):
        idxp = (idx[p * EP:(p + 1) * EP].reshape(CP, 1, CH) + off) \
            .reshape(2 * CP, CH)
        gs.append(_sc_gather(table, idxp))
    accs = None
    for p in range(NP):
        g01, g23 = gs[p]
        out = _tc_piece(p, accs, evt, g01, g23, edge_0, e1v, weights)
        accs = tuple(out)
    out0, out1v = accs
    return (out0, jnp.transpose(out1v, (1, 2, 0)))


# speedup vs baseline: 1.2361x; 1.0625x over previous
import functools

import jax
import jax.numpy as jnp
from jax import lax
from jax.experimental import pallas as pl
from jax.experimental.pallas import tpu as pltpu
from jax.experimental.pallas import tpu_sc as plsc

N = 10000
E = 160000
DIM = 128
NB = 8
RC = 5.0
BE = 3200
NP = 5
EP = E // NP
BPP = EP // BE

_NC, _NS = 2, 16
_NW = _NC * _NS
CH = 128
CP = EP // CH
TPW = -(-CP // _NW)


def _sc_gather(table, idxp):
    mesh = plsc.VectorSubcoreMesh(core_axis_name="c", subcore_axis_name="s")
    out = jax.ShapeDtypeStruct((EP, DIM), jnp.float32)

    @functools.partial(
        pl.kernel,
        mesh=mesh,
        out_type=[out, out],
        scratch_types=[
            pltpu.VMEM((2, CH), jnp.int32),
            [[pltpu.VMEM((CH, DIM), jnp.float32) for _ in range(2)]
             for _ in range(2)],
            pltpu.SemaphoreType.DMA,
            [pltpu.SemaphoreType.DMA for _ in range(2)],
        ],
    )
    def gather_kernel(tab_hbm, idx_hbm, g0, g1, idx_v, rows, gsem, wsems):
        outs = (g0, g1)
        wid = lax.axis_index("s") * _NC + lax.axis_index("c")

        def wcopies(t):
            b = t % 2
            chunk = t * _NW + wid
            return [pltpu.make_async_copy(
                rows[b][a], outs[a].at[pl.ds(chunk * CH, CH)], wsems[b])
                for a in range(2)]

        for t in range(TPW):
            chunk = t * _NW + wid

            @pl.when(chunk < CP)
            def _():
                if t >= 2:
                    for c in wcopies(t - 2):
                        c.wait()
                pltpu.sync_copy(idx_hbm.at[pl.ds(2 * chunk, 2)], idx_v)
                b = t % 2
                cps = [
                    pltpu.async_copy(
                        tab_hbm.at[idx_v.at[a, pl.ds(h * (CH // 2), CH // 2)]],
                        rows[b][a].at[pl.ds(h * (CH // 2), CH // 2)], gsem)
                    for a in range(2) for h in range(2)
                ]
                for c in cps:
                    c.wait()
                for c in wcopies(t):
                    c.start()

        for u in range(max(0, TPW - 3), TPW):
            skipped = (u * _NW + wid < CP)
            if u + 2 < TPW:
                skipped = skipped & ((u + 2) * _NW + wid >= CP)

            @pl.when(skipped)
            def _():
                for c in wcopies(u):
                    c.wait()

    return gather_kernel(table, idxp)


def _unpack_pair(packed):
    iv = lax.bitcast_convert_type(packed, jnp.uint32)
    hi = lax.bitcast_convert_type(iv & jnp.uint32(0xFFFF0000), jnp.float32)
    lo = lax.bitcast_convert_type(iv << 16, jnp.float32)
    return hi, lo


def _tc_body(evt_ref, g01_ref, g23_ref, e0_ref, e1_ref,
             wr000, wr110, wr011, wr101, wm01, wm02, wm1c, wm1g, wg,
             out0_ref, out1_ref):
    f32 = jnp.float32
    bf16 = jnp.bfloat16
    dimn = (((0,), (0,)), ((), ()))
    dott = lambda a, b: lax.dot_general(a, b, dimn, preferred_element_type=f32)
    dot = lambda a, b: jnp.dot(a.astype(bf16), b, preferred_element_type=f32)

    evt = evt_ref[...]
    d2 = jnp.sum(evt * evt, axis=0, keepdims=True)
    d = jnp.sqrt(d2)
    rinv = 1.0 / (d + 1e-9)
    rt = evt * rinv
    centers = lax.broadcasted_iota(jnp.int32, (NB, 1), 0).astype(f32) \
        * (RC / (NB - 1))
    fc = 0.5 * (jnp.cos(jnp.pi * jnp.clip(d * (1.0 / RC), 0.0, 1.0)) + 1.0)
    rbft = jnp.exp(-2.0 * (d - centers) ** 2) * fc

    f000 = dott(rbft, wr000[...])
    f101 = dott(rbft, wr101[...])
    f110r = [dott(rbft * rt[a:a + 1], wr110[...]) for a in range(3)]
    f011r = [dott(rbft * rt[a:a + 1], wr011[...]) for a in range(3)]

    x0, x1a = _unpack_pair(g01_ref[...])
    x1b, x1c = _unpack_pair(g23_ref[...])
    x1 = (x1a, x1b, x1c)
    m0 = f000 * x0 + f110r[0] * x1[0] + f110r[1] * x1[1] + f110r[2] * x1[2]
    m1 = [f101 * x1[a] + f011r[a] * x0 for a in range(3)]

    s0 = dot(m0, wm01[...]) + dot(m0 * m0, wm02[...])
    gmix = dot(m0, wm1g[...])
    s1 = [dot(m1[a], wm1c[...]) for a in range(3)]

    n0 = s0 * jax.nn.sigmoid(s0)
    gate = jax.nn.sigmoid(dot(s0, wg[...]))
    out0_ref[...] = e0_ref[...] + n0
    for a in range(3):
        n1a = (s1[a] + gmix * m1[a]) * gate
        out1_ref[a] = e1_ref[a] + n1a


def _tc_body_acc(acc0_ref, acc1_ref, *args):
    del acc0_ref, acc1_ref
    _tc_body(*args)


def _tc_piece(p, accs, evt, g01, g23, edge_0, e1v, weights):
    row_f = lambda i, p=p: (p * BPP + i, 0)
    row_p = lambda i: (i, 0)
    comp = lambda i, p=p: (0, p * BPP + i, 0)
    fixed = lambda i: (0, 0)
    hbm = pl.BlockSpec(memory_space=pltpu.MemorySpace.HBM)
    in_specs = [
        pl.BlockSpec((3, BE), lambda i, p=p: (0, p * BPP + i)),
        pl.BlockSpec((BE, DIM), row_p),
        pl.BlockSpec((BE, DIM), row_p),
        pl.BlockSpec((BE, DIM), row_f),
        pl.BlockSpec((3, BE, DIM), comp),
    ] + [pl.BlockSpec((NB, DIM), fixed)] * 4 + \
        [pl.BlockSpec((DIM, DIM), fixed)] * 5
    operands = (evt, g01, g23, edge_0, e1v) + tuple(weights)
    kwargs = {}
    body = _tc_body
    if accs is not None:
        in_specs = [hbm, hbm] + in_specs
        operands = accs + operands
        kwargs["input_output_aliases"] = {0: 0, 1: 1}
        body = _tc_body_acc
    return pl.pallas_call(
        body,
        grid=(BPP,),
        in_specs=in_specs,
        out_specs=[
            pl.BlockSpec((BE, DIM), row_f),
            pl.BlockSpec((3, BE, DIM), comp),
        ],
        out_shape=[
            jax.ShapeDtypeStruct((E, DIM), jnp.float32),
            jax.ShapeDtypeStruct((3, E, DIM), jnp.float32),
        ],
        **kwargs,
    )(*operands)


def kernel(node_0, node_1, edge_0, edge_1, edge_index, edge_vec,
           Wr_000, Wr_110, Wr_011, Wr_101, Wm0_1, Wm0_2, Wm1_c, Wm1_g, Wg):
    idx = edge_index[0].astype(jnp.int32)
    off = jnp.array([0, N], dtype=jnp.int32).reshape(1, 2, 1)
    slabs = jnp.transpose(node_1, (2, 0, 1))
    e1v = jnp.transpose(edge_1, (2, 0, 1))
    evt = jnp.transpose(edge_vec)
    bf = jnp.bfloat16
    weights = (Wr_000, Wr_110, Wr_011, Wr_101,
               Wm0_1.astype(bf), Wm0_2.astype(bf), Wm1_c.astype(bf),
               Wm1_g.astype(bf), Wg.astype(bf))

    def _pack(hi, lo):
        hb = lax.bitcast_convert_type(hi.astype(jnp.bfloat16), jnp.uint16)
        lb = lax.bitcast_convert_type(lo.astype(jnp.bfloat16), jnp.uint16)
        w = (hb.astype(jnp.uint32) << 16) | lb.astype(jnp.uint32)
        return lax.bitcast_convert_type(w, jnp.float32)

    table = jnp.concatenate(
        [_pack(node_0, slabs[0]), _pack(slabs[1], slabs[2])], axis=0)

    gs = []
    for p in range(NP):
        idxp = (idx[p * EP:(p + 1) * EP].reshape(CP, 1, CH) + off) \
            .reshape(2 * CP, CH)
        gs.append(_sc_gather(table, idxp))
    accs = None
    for p in range(NP):
        g01, g23 = gs[p]
        out = _tc_piece(p, accs, evt, g01, g23, edge_0, e1v, weights)
        accs = tuple(out)
    out0, out1v = accs
    return (out0, jnp.transpose(out1v, (1, 2, 0)))
